# baseline (speedup 1.0000x reference)
import functools

import jax
import jax.numpy as jnp
from jax import lax
from jax.experimental import pallas as pl
from jax.experimental.pallas import tpu as pltpu
from jax.experimental.pallas import tpu_sc as plsc

_N = 10000
_E = 320000
_C = 128


def _mm_body(nadd, act, two_out, *refs):
    x = refs[0][...]
    for i in range(nadd):
        x = x + refs[1 + i][...]
    w = refs[1 + nadd][...]
    b = refs[2 + nadd][...]
    y = jnp.dot(x, w, preferred_element_type=jnp.float32) + b
    if act == "relu":
        y = jnp.maximum(y, 0.0)
    refs[3 + nadd][...] = y.astype(refs[3 + nadd].dtype)
    if two_out:
        refs[4 + nadd][...] = x


def _mm(x, w, b, act=None, extra_adds=(), block_rows=1000, out_sum=False,
        out_dtype=jnp.float32):
    n, k = x.shape
    m = w.shape[1]
    nadd = len(extra_adds)
    grid = (n // block_rows,)
    row_spec = pl.BlockSpec((block_rows, k), lambda i: (i, 0))
    in_specs = [row_spec] * (1 + nadd) + [
        pl.BlockSpec((k, m), lambda i: (0, 0)),
        pl.BlockSpec((1, m), lambda i: (0, 0)),
    ]
    out_spec = pl.BlockSpec((block_rows, m), lambda i: (i, 0))
    out_shape = jax.ShapeDtypeStruct((n, m), out_dtype)
    if out_sum:
        out_shape = (out_shape, jax.ShapeDtypeStruct((n, k), jnp.float32))
        out_specs = (out_spec, row_spec)
    else:
        out_specs = out_spec
    fn = pl.pallas_call(
        functools.partial(_mm_body, nadd, act, out_sum),
        grid=grid,
        in_specs=in_specs,
        out_specs=out_specs,
        out_shape=out_shape,
    )
    return fn(x, *extra_adds, w, b.reshape(1, m))


def _mm3_body(x_ref, w_ref, b_ref, o1_ref, o2_ref, o3_ref):
    y = jnp.dot(x_ref[...], w_ref[...],
                preferred_element_type=jnp.float32) + b_ref[...]
    m = o1_ref.shape[1]
    o1_ref[...] = y[:, :m]
    o2_ref[...] = y[:, m:2 * m]
    o3_ref[...] = y[:, 2 * m:]


def _mm3(x, ws, bs, block_rows=1000):
    n, k = x.shape
    m = ws[0].shape[1]
    w3 = jnp.concatenate(ws, axis=1)
    b3 = jnp.concatenate(bs).reshape(1, 3 * m)
    grid = (n // block_rows,)
    out_spec = pl.BlockSpec((block_rows, m), lambda i: (i, 0))
    fn = pl.pallas_call(
        _mm3_body,
        grid=grid,
        in_specs=[pl.BlockSpec((block_rows, k), lambda i: (i, 0)),
                  pl.BlockSpec((k, 3 * m), lambda i: (0, 0)),
                  pl.BlockSpec((1, 3 * m), lambda i: (0, 0))],
        out_specs=(out_spec, out_spec, out_spec),
        out_shape=tuple(jax.ShapeDtypeStruct((n, m), jnp.float32)
                        for _ in range(3)),
    )
    return fn(x, w3, b3)


def _stats_body(has_resid, *refs):
    if has_resid:
        t = refs[0][...] + refs[1][...]
        refs[2][...] = t
        s_ref = refs[3]
    else:
        t = refs[0][...]
        s_ref = refs[1]

    @pl.when(pl.program_id(0) == 0)
    def _():
        s_ref[...] = jnp.zeros_like(s_ref)

    ps = jnp.sum(t, axis=0, keepdims=True)
    pq = jnp.sum(t * t, axis=0, keepdims=True)
    pad = jnp.zeros((6, t.shape[1]), jnp.float32)
    s_ref[...] += jnp.concatenate([ps, pq, pad], axis=0)


def _add_stats(a, r=None, block_rows=1000):
    n, c = a.shape
    grid = (n // block_rows,)
    row_spec = pl.BlockSpec((block_rows, c), lambda i: (i, 0))
    s_spec = pl.BlockSpec((8, c), lambda i: (0, 0))
    s_shape = jax.ShapeDtypeStruct((8, c), jnp.float32)
    if r is None:
        fn = pl.pallas_call(
            functools.partial(_stats_body, False),
            grid=grid,
            in_specs=[row_spec],
            out_specs=s_spec,
            out_shape=s_shape,
        )
        return a, fn(a)
    fn = pl.pallas_call(
        functools.partial(_stats_body, True),
        grid=grid,
        in_specs=[row_spec, row_spec],
        out_specs=(row_spec, s_spec),
        out_shape=(jax.ShapeDtypeStruct((n, c), jnp.float32), s_shape),
    )
    return fn(a, r)


def _bn_body(relu, n, *refs):
    x_ref, s_ref, g_ref, b_ref, o_ref = refs
    s = s_ref[...]
    mu = s[0:1, :] / n
    var = s[1:2, :] / n - mu * mu
    rstd = lax.rsqrt(var + 1e-5)
    y = (x_ref[...] - mu) * (rstd * g_ref[...]) + b_ref[...]
    if relu:
        y = jnp.maximum(y, 0.0)
    o_ref[...] = y


def _bn_apply(x, s, g, b, relu=False, block_rows=1000):
    n, c = x.shape
    grid = (n // block_rows,)
    row_spec = pl.BlockSpec((block_rows, c), lambda i: (i, 0))
    vec_spec = pl.BlockSpec((1, c), lambda i: (0, 0))
    fn = pl.pallas_call(
        lambda *refs: _bn_body(relu, float(n), *refs),
        grid=grid,
        in_specs=[row_spec, pl.BlockSpec((8, c), lambda i: (0, 0)),
                  vec_spec, vec_spec],
        out_specs=row_spec,
        out_shape=jax.ShapeDtypeStruct((n, c), jnp.float32),
    )
    return fn(x, s, g.reshape(1, c), b.reshape(1, c))


def _attn_body(scale, q_ref, k_ref, v_ref, o_ref):
    q = q_ref[0].astype(jnp.bfloat16)
    k = k_ref[0].astype(jnp.bfloat16)
    s = lax.dot_general(q, k, (((1,), (1,)), ((), ())),
                        preferred_element_type=jnp.float32) * scale
    m = jnp.max(s, axis=1, keepdims=True)
    p = jnp.exp(s - m)
    denom = jnp.sum(p, axis=1, keepdims=True)
    o = lax.dot_general(p.astype(jnp.bfloat16),
                        v_ref[0].astype(jnp.bfloat16),
                        (((1,), (0,)), ((), ())),
                        preferred_element_type=jnp.float32)
    o_ref[0] = o / denom


def _attention(q, k, v, heads, block_q=400):
    n, c = q.shape
    dh = c // heads
    scale = float(dh) ** -0.5
    qh = q.reshape(n, heads, dh).transpose(1, 0, 2)
    kh = k.reshape(n, heads, dh).transpose(1, 0, 2)
    vh = v.reshape(n, heads, dh).transpose(1, 0, 2)
    grid = (heads, n // block_q)
    q_spec = pl.BlockSpec((1, block_q, dh), lambda h, i: (h, i, 0))
    kv_spec = pl.BlockSpec((1, n, dh), lambda h, i: (h, 0, 0))
    fn = pl.pallas_call(
        functools.partial(_attn_body, scale),
        grid=grid,
        in_specs=[q_spec, kv_spec, kv_spec],
        out_specs=q_spec,
        out_shape=jax.ShapeDtypeStruct((heads, n, dh), jnp.float32),
    )
    oh = fn(qh, kh, vh)
    return oh.transpose(1, 0, 2).reshape(n, c)


_CH = 80
_ROWS_PER_SUB = 624


def _gine_body(x_hbm, ea_hbm, src_hbm, dst_hbm, z_hbm, out_hbm,
               sidx0, didx0, rows0, eab0, msg0, sidx1, didx1, rows1, eab1,
               msg1, agg_sh, lsem0, gsem0, lsem1, gsem1):
    cid = lax.axis_index("c")
    sid = lax.axis_index("s")
    wid = sid * 2 + cid
    e0 = wid * (_E // 32)
    bufA = (sidx0, didx0, rows0, eab0, msg0, lsem0, gsem0)
    bufB = (sidx1, didx1, rows1, eab1, msg1, lsem1, gsem1)

    def issue_loads(c, bs):
        sidx, didx, rows, eab, msg, lsem, gsem = bs
        base = e0 + c * _CH
        pltpu.async_copy(src_hbm.at[pl.ds(base, _CH)], sidx, lsem)
        pltpu.async_copy(dst_hbm.at[pl.ds(base, _CH)], didx, lsem)
        pltpu.async_copy(ea_hbm.at[pl.ds(base, _CH)], eab, lsem)

    def wait_loads(c, bs):
        sidx, didx, rows, eab, msg, lsem, gsem = bs
        base = e0 + c * _CH
        pltpu.make_async_copy(src_hbm.at[pl.ds(base, _CH)], sidx, lsem).wait()
        pltpu.make_async_copy(dst_hbm.at[pl.ds(base, _CH)], didx, lsem).wait()
        pltpu.make_async_copy(ea_hbm.at[pl.ds(base, _CH)], eab, lsem).wait()

    def issue_gather(bs):
        sidx, didx, rows, eab, msg, lsem, gsem = bs
        pltpu.async_copy(x_hbm.at[sidx], rows, gsem)

    def wait_gather(bs):
        sidx, didx, rows, eab, msg, lsem, gsem = bs
        pltpu.make_async_copy(x_hbm.at[sidx], rows, gsem).wait()

    def unpack2(ref2d, r, j):
        wi = ref2d[r, pl.ds(j * 16, 16)]
        return plsc.unpack(plsc.bitcast(wi, jnp.bfloat16),
                           format=plsc.PackFormat.INTERLEAVED)

    def compute_scatter(bs):
        sidx, didx, rows, eab, msg, lsem, gsem = bs

        def edge4(i, carry):
            for u in range(4):
                r = i * 4 + u
                for j in range(4):
                    lx, hx = unpack2(rows, r, j)
                    le, he = unpack2(eab, r, j)
                    msg[r, pl.ds(j * 32, 16)] = jnp.maximum(lx + le, 0.0)
                    msg[r, pl.ds(j * 32 + 16, 16)] = jnp.maximum(hx + he, 0.0)
            return carry

        lax.fori_loop(0, _CH // 4, edge4, 0)
        pltpu.sync_copy(msg, agg_sh.at[didx], add=True)

    r0 = sid * _ROWS_PER_SUB
    tail = 16 * _ROWS_PER_SUB
    pltpu.sync_copy(z_hbm.at[pl.ds(r0, _ROWS_PER_SUB)],
                    agg_sh.at[pl.ds(r0, _ROWS_PER_SUB)])

    @pl.when(sid == 15)
    def _():
        pltpu.sync_copy(z_hbm.at[pl.ds(tail, _N - tail)],
                        agg_sh.at[pl.ds(tail, _N - tail)])

    plsc.subcore_barrier()

    nchunks = (_E // 32) // _CH
    npairs = (nchunks - 1) // 2

    issue_loads(0, bufA)
    wait_loads(0, bufA)
    issue_gather(bufA)
    issue_loads(1, bufB)

    def pair(h, carry):
        c = 2 * h
        wait_loads(c + 1, bufB)
        issue_gather(bufB)
        wait_gather(bufA)
        compute_scatter(bufA)
        issue_loads(c + 2, bufA)
        wait_loads(c + 2, bufA)
        issue_gather(bufA)
        wait_gather(bufB)
        compute_scatter(bufB)

        @pl.when(h < npairs - 1)
        def _():
            issue_loads(c + 3, bufB)

        return carry

    lax.fori_loop(0, npairs, pair, 0)
    wait_gather(bufA)
    compute_scatter(bufA)
    plsc.subcore_barrier()

    pltpu.sync_copy(agg_sh.at[pl.ds(r0, _ROWS_PER_SUB)],
                    out_hbm.at[pl.ds(cid * _N + r0, _ROWS_PER_SUB)])

    @pl.when(sid == 15)
    def _():
        pltpu.sync_copy(agg_sh.at[pl.ds(tail, _N - tail)],
                        out_hbm.at[pl.ds(cid * _N + tail, _N - tail)])


def _perm_idx():
    idx = []
    for g in range(4):
        for i in range(16):
            idx.append(g * 32 + i)
            idx.append(g * 32 + 16 + i)
    return tuple(idx)


_PERM = _perm_idx()


def _gine_sc(xbf, eabf, src, dst):
    c = _C
    x = lax.bitcast_convert_type(xbf.reshape(_N, c // 2, 2), jnp.int32)
    ea = lax.bitcast_convert_type(eabf.reshape(_E, c // 2, 2), jnp.int32)
    mesh = plsc.VectorSubcoreMesh(core_axis_name="c", subcore_axis_name="s")
    fn = functools.partial(
        pl.kernel,
        mesh=mesh,
        compiler_params=pltpu.CompilerParams(needs_layout_passes=False,
                                             use_tc_tiling_on_sc=False),
        out_type=jax.ShapeDtypeStruct((2 * _N, c), jnp.float32),
        scratch_types=[
            pltpu.VMEM((_CH,), jnp.int32),
            pltpu.VMEM((_CH,), jnp.int32),
            pltpu.VMEM((_CH, c // 2), jnp.int32),
            pltpu.VMEM((_CH, c // 2), jnp.int32),
            pltpu.VMEM((_CH, c), jnp.float32),
            pltpu.VMEM((_CH,), jnp.int32),
            pltpu.VMEM((_CH,), jnp.int32),
            pltpu.VMEM((_CH, c // 2), jnp.int32),
            pltpu.VMEM((_CH, c // 2), jnp.int32),
            pltpu.VMEM((_CH, c), jnp.float32),
            pltpu.VMEM_SHARED((_N, c), jnp.float32),
            pltpu.SemaphoreType.DMA,
            pltpu.SemaphoreType.DMA,
            pltpu.SemaphoreType.DMA,
            pltpu.SemaphoreType.DMA,
        ],
    )(_gine_body)
    zeros = jnp.zeros((_N, c), jnp.float32)
    return fn(x, ea, src, dst, zeros)


def _gps_layer(x, xbf, ea, src, dst, p, heads):
    aggs = _gine_sc(xbf, ea, src, dst)
    q, k, v = _mm3(x, (p["Wq"], p["Wk"], p["Wv"]), (p["bq"], p["bk"], p["bv"]))
    ao = _attention(q, k, v, heads)
    ha = _mm(ao, p["Wo"], p["bo"])
    t2, s2 = _add_stats(ha, x)
    han = _bn_apply(t2, s2, p["n2_g"], p["n2_b"])

    y1 = _mm(x, p["W1"], p["b1"], act="relu",
             extra_adds=(aggs[:_N], aggs[_N:]))
    h2 = _mm(y1, p["W2"], p["b2"])
    t1, s1 = _add_stats(h2, x)
    hn = _bn_apply(t1, s1, p["n1_g"], p["n1_b"])

    y, out = _mm(hn, p["mW1"], p["mb1"], act="relu", extra_adds=(han,),
                 out_sum=True)
    r2 = _mm(y, p["mW2"], p["mb2"])
    t3, s3 = _add_stats(r2, out)
    return _bn_apply(t3, s3, p["n3_g"], p["n3_b"])


def _bn_relu(x, g, b):
    _, s = _add_stats(x)
    return _bn_apply(x, s, g, b, relu=True)


def kernel(x, edge_attr, params, edge_index):
    p1, p2 = params["gps1"], params["gps2"]
    src = edge_index[0]
    dst = edge_index[1]
    perm = jnp.asarray(_PERM)
    pmat = jnp.eye(_C, dtype=jnp.float32)[:, perm]
    zb = jnp.zeros((_C,), jnp.float32)
    ea1 = _mm(edge_attr, p1["We"][:, perm], p1["be"][perm],
              block_rows=4000, out_dtype=jnp.bfloat16)
    ea2 = _mm(edge_attr, p2["We"][:, perm], p2["be"][perm],
              block_rows=4000, out_dtype=jnp.bfloat16)
    xbf = _mm(x, pmat, zb, out_dtype=jnp.bfloat16)
    h = _gps_layer(x, xbf, ea1, src, dst, p1, heads=2)
    h = _bn_relu(h, params["bn1_g"], params["bn1_b"])
    h = _mm(h, params["lin1_W"], params["lin1_b"])
    h = _bn_relu(h, params["bn2_g"], params["bn2_b"])
    hbf = _mm(h, pmat, zb, out_dtype=jnp.bfloat16)
    h = _gps_layer(h, hbf, ea2, src, dst, p2, heads=1)
    h = _bn_relu(h, params["bn2_g"], params["bn2_b"])
    return _mm(h, params["lin2_W"], params["lin2_b"])


# speedup vs baseline: 2.3672x; 2.3672x over previous
import functools

import jax
import jax.numpy as jnp
from jax import lax
from jax.experimental import pallas as pl
from jax.experimental.pallas import tpu as pltpu
from jax.experimental.pallas import tpu_sc as plsc

_N = 10000
_E = 320000
_C = 128


def _bn_math(x, s, g, b, n_rows, relu):
    mu = s[0:1, :] / n_rows
    var = s[1:2, :] / n_rows - mu * mu
    rstd = lax.rsqrt(var + 1e-5)
    y = (x - mu) * (rstd * g) + b
    if relu:
        y = jnp.maximum(y, 0.0)
    return y


def _acc_stats(s_ref, y):
    @pl.when(pl.program_id(0) == 0)
    def _():
        s_ref[...] = jnp.zeros_like(s_ref)

    ps = jnp.sum(y, axis=0, keepdims=True)
    pq = jnp.sum(y * y, axis=0, keepdims=True)
    pad = jnp.zeros((6, y.shape[1]), jnp.float32)
    s_ref[...] += jnp.concatenate([ps, pq, pad], axis=0)


def _mm_body(nadd, npost, act, two_out, has_bn, bn_relu, out_stats, n_rows,
             *refs):
    i = 0
    x = refs[i][...]
    i += 1
    for _ in range(nadd):
        x = x + refs[i][...]
        i += 1
    if has_bn:
        x = _bn_math(x, refs[i][...], refs[i + 1][...], refs[i + 2][...],
                     n_rows, bn_relu)
        i += 3
    w = refs[i][...]
    b = refs[i + 1][...]
    i += 2
    y = jnp.dot(x, w, preferred_element_type=jnp.float32) + b
    if act == "relu":
        y = jnp.maximum(y, 0.0)
    for _ in range(npost):
        y = y + refs[i][...]
        i += 1
    refs[i][...] = y.astype(refs[i].dtype)
    i += 1
    if two_out:
        refs[i][...] = x
        i += 1
    if out_stats:
        _acc_stats(refs[i], y)


def _mm(x, w, b, act=None, extra_adds=(), post_adds=(), pre_bn=None,
        block_rows=1000, out_sum=False, out_stats=False,
        out_dtype=jnp.float32):
    n, k = x.shape
    m = w.shape[1]
    nadd = len(extra_adds)
    npost = len(post_adds)
    grid = (n // block_rows,)
    row_spec = pl.BlockSpec((block_rows, k), lambda i: (i, 0))
    out_spec = pl.BlockSpec((block_rows, m), lambda i: (i, 0))
    stat_spec = pl.BlockSpec((8, m), lambda i: (0, 0))
    vec_spec = pl.BlockSpec((1, m), lambda i: (0, 0))
    in_specs = [row_spec] * (1 + nadd)
    args = [x, *extra_adds]
    if pre_bn is not None:
        s, g, bb, bn_relu = pre_bn
        in_specs += [pl.BlockSpec((8, k), lambda i: (0, 0)),
                     pl.BlockSpec((1, k), lambda i: (0, 0)),
                     pl.BlockSpec((1, k), lambda i: (0, 0))]
        args += [s, g.reshape(1, k), bb.reshape(1, k)]
    else:
        bn_relu = False
    in_specs += [pl.BlockSpec((k, m), lambda i: (0, 0)), vec_spec]
    args += [w, b.reshape(1, m)]
    in_specs += [out_spec] * npost
    args += list(post_adds)
    out_shape = [jax.ShapeDtypeStruct((n, m), out_dtype)]
    out_specs = [out_spec]
    if out_sum:
        out_shape.append(jax.ShapeDtypeStruct((n, k), jnp.float32))
        out_specs.append(row_spec)
    if out_stats:
        out_shape.append(jax.ShapeDtypeStruct((8, m), jnp.float32))
        out_specs.append(stat_spec)
    fn = pl.pallas_call(
        functools.partial(_mm_body, nadd, npost, act, out_sum,
                          pre_bn is not None, bn_relu, out_stats, float(n)),
        grid=grid,
        in_specs=in_specs,
        out_specs=tuple(out_specs),
        out_shape=tuple(out_shape),
    )
    out = fn(*args)
    return out if (out_sum or out_stats) else out[0]


def _mm3_body(x_ref, w_ref, b_ref, o1_ref, o2_ref, o3_ref):
    y = jnp.dot(x_ref[...], w_ref[...],
                preferred_element_type=jnp.float32) + b_ref[...]
    m = o1_ref.shape[1]
    o1_ref[...] = y[:, :m]
    o2_ref[...] = y[:, m:2 * m]
    o3_ref[...] = y[:, 2 * m:]


def _mm3(x, ws, bs, block_rows=1000):
    n, k = x.shape
    m = ws[0].shape[1]
    w3 = jnp.concatenate(ws, axis=1)
    b3 = jnp.concatenate(bs).reshape(1, 3 * m)
    grid = (n // block_rows,)
    out_spec = pl.BlockSpec((block_rows, m), lambda i: (i, 0))
    fn = pl.pallas_call(
        _mm3_body,
        grid=grid,
        in_specs=[pl.BlockSpec((block_rows, k), lambda i: (i, 0)),
                  pl.BlockSpec((k, 3 * m), lambda i: (0, 0)),
                  pl.BlockSpec((1, 3 * m), lambda i: (0, 0))],
        out_specs=(out_spec, out_spec, out_spec),
        out_shape=tuple(jax.ShapeDtypeStruct((n, m), jnp.float32)
                        for _ in range(3)),
    )
    return fn(x, w3, b3)


def _stats_body(has_resid, *refs):
    if has_resid:
        t = refs[0][...] + refs[1][...]
        refs[2][...] = t
        s_ref = refs[3]
    else:
        t = refs[0][...]
        s_ref = refs[1]

    @pl.when(pl.program_id(0) == 0)
    def _():
        s_ref[...] = jnp.zeros_like(s_ref)

    ps = jnp.sum(t, axis=0, keepdims=True)
    pq = jnp.sum(t * t, axis=0, keepdims=True)
    pad = jnp.zeros((6, t.shape[1]), jnp.float32)
    s_ref[...] += jnp.concatenate([ps, pq, pad], axis=0)


def _add_stats(a, r=None, block_rows=1000):
    n, c = a.shape
    grid = (n // block_rows,)
    row_spec = pl.BlockSpec((block_rows, c), lambda i: (i, 0))
    s_spec = pl.BlockSpec((8, c), lambda i: (0, 0))
    s_shape = jax.ShapeDtypeStruct((8, c), jnp.float32)
    if r is None:
        fn = pl.pallas_call(
            functools.partial(_stats_body, False),
            grid=grid,
            in_specs=[row_spec],
            out_specs=s_spec,
            out_shape=s_shape,
        )
        return a, fn(a)
    fn = pl.pallas_call(
        functools.partial(_stats_body, True),
        grid=grid,
        in_specs=[row_spec, row_spec],
        out_specs=(row_spec, s_spec),
        out_shape=(jax.ShapeDtypeStruct((n, c), jnp.float32), s_shape),
    )
    return fn(a, r)


def _bn_body(relu, n, out_stats, *refs):
    x_ref, s_ref, g_ref, b_ref, o_ref = refs[:5]
    y = _bn_math(x_ref[...], s_ref[...], g_ref[...], b_ref[...], n, relu)
    o_ref[...] = y
    if out_stats:
        _acc_stats(refs[5], y)


def _bn_apply(x, s, g, b, relu=False, out_stats=False, block_rows=1000):
    n, c = x.shape
    grid = (n // block_rows,)
    row_spec = pl.BlockSpec((block_rows, c), lambda i: (i, 0))
    vec_spec = pl.BlockSpec((1, c), lambda i: (0, 0))
    stat_spec = pl.BlockSpec((8, c), lambda i: (0, 0))
    out_specs = [row_spec]
    out_shape = [jax.ShapeDtypeStruct((n, c), jnp.float32)]
    if out_stats:
        out_specs.append(stat_spec)
        out_shape.append(jax.ShapeDtypeStruct((8, c), jnp.float32))
    fn = pl.pallas_call(
        lambda *refs: _bn_body(relu, float(n), out_stats, *refs),
        grid=grid,
        in_specs=[row_spec, stat_spec, vec_spec, vec_spec],
        out_specs=tuple(out_specs),
        out_shape=tuple(out_shape),
    )
    out = fn(x, s, g.reshape(1, c), b.reshape(1, c))
    return out if out_stats else out[0]


def _attn_body(scale, q_ref, k_ref, v_ref, o_ref):
    q = q_ref[0].astype(jnp.bfloat16)
    k = k_ref[0].astype(jnp.bfloat16)
    s = lax.dot_general(q, k, (((1,), (1,)), ((), ())),
                        preferred_element_type=jnp.float32) * scale
    m = jnp.max(s, axis=1, keepdims=True)
    p = jnp.exp(s - m)
    denom = jnp.sum(p, axis=1, keepdims=True)
    o = lax.dot_general(p.astype(jnp.bfloat16),
                        v_ref[0].astype(jnp.bfloat16),
                        (((1,), (0,)), ((), ())),
                        preferred_element_type=jnp.float32)
    o_ref[0] = o / denom


def _attention(q, k, v, heads, block_q=400):
    n, c = q.shape
    dh = c // heads
    scale = float(dh) ** -0.5
    qh = q.reshape(n, heads, dh).transpose(1, 0, 2)
    kh = k.reshape(n, heads, dh).transpose(1, 0, 2)
    vh = v.reshape(n, heads, dh).transpose(1, 0, 2)
    grid = (heads, n // block_q)
    q_spec = pl.BlockSpec((1, block_q, dh), lambda h, i: (h, i, 0))
    kv_spec = pl.BlockSpec((1, n, dh), lambda h, i: (h, 0, 0))
    fn = pl.pallas_call(
        functools.partial(_attn_body, scale),
        grid=grid,
        in_specs=[q_spec, kv_spec, kv_spec],
        out_specs=q_spec,
        out_shape=jax.ShapeDtypeStruct((heads, n, dh), jnp.float32),
    )
    oh = fn(qh, kh, vh)
    return oh.transpose(1, 0, 2).reshape(n, c)


_CH = 80
_ROWS_PER_SUB = 624


def _gine_body(x_hbm, ea_hbm, src_hbm, dst_hbm, z_hbm, out_hbm,
               sidx0, didx0, rows0, eab0, msg0, sidx1, didx1, rows1, eab1,
               msg1, agg_sh, lsem0, gsem0, lsem1, gsem1):
    cid = lax.axis_index("c")
    sid = lax.axis_index("s")
    wid = sid * 2 + cid
    e0 = wid * (_E // 32)
    bufA = (sidx0, didx0, rows0, eab0, msg0, lsem0, gsem0)
    bufB = (sidx1, didx1, rows1, eab1, msg1, lsem1, gsem1)

    def issue_loads(c, bs):
        sidx, didx, rows, eab, msg, lsem, gsem = bs
        base = e0 + c * _CH
        pltpu.async_copy(src_hbm.at[pl.ds(base, _CH)], sidx, lsem)
        pltpu.async_copy(dst_hbm.at[pl.ds(base, _CH)], didx, lsem)
        pltpu.async_copy(ea_hbm.at[pl.ds(base, _CH)], eab, lsem)

    def wait_loads(c, bs):
        sidx, didx, rows, eab, msg, lsem, gsem = bs
        base = e0 + c * _CH
        pltpu.make_async_copy(src_hbm.at[pl.ds(base, _CH)], sidx, lsem).wait()
        pltpu.make_async_copy(dst_hbm.at[pl.ds(base, _CH)], didx, lsem).wait()
        pltpu.make_async_copy(ea_hbm.at[pl.ds(base, _CH)], eab, lsem).wait()

    def issue_gather(bs):
        sidx, didx, rows, eab, msg, lsem, gsem = bs
        pltpu.async_copy(x_hbm.at[sidx], rows, gsem)

    def wait_gather(bs):
        sidx, didx, rows, eab, msg, lsem, gsem = bs
        pltpu.make_async_copy(x_hbm.at[sidx], rows, gsem).wait()

    def compute_scatter(bs):
        sidx, didx, rows, eab, msg, lsem, gsem = bs

        def edge4(i, carry):
            for u in range(4):
                r = i * 4 + u
                for j in range(8):
                    sl = pl.ds(j * 16, 16)
                    rows[r, sl] = jnp.maximum(rows[r, sl] + eab[r, sl], 0.0)
            return carry

        lax.fori_loop(0, _CH // 4, edge4, 0)
        pltpu.sync_copy(rows, agg_sh.at[didx], add=True)

    r0 = sid * _ROWS_PER_SUB
    tail = 16 * _ROWS_PER_SUB
    pltpu.sync_copy(z_hbm.at[pl.ds(r0, _ROWS_PER_SUB)],
                    agg_sh.at[pl.ds(r0, _ROWS_PER_SUB)])

    @pl.when(sid == 15)
    def _():
        pltpu.sync_copy(z_hbm.at[pl.ds(tail, _N - tail)],
                        agg_sh.at[pl.ds(tail, _N - tail)])

    plsc.subcore_barrier()

    nchunks = (_E // 32) // _CH
    npairs = (nchunks - 1) // 2

    issue_loads(0, bufA)
    wait_loads(0, bufA)
    issue_gather(bufA)
    issue_loads(1, bufB)

    def pair(h, carry):
        c = 2 * h
        wait_loads(c + 1, bufB)
        issue_gather(bufB)
        wait_gather(bufA)
        compute_scatter(bufA)
        issue_loads(c + 2, bufA)
        wait_loads(c + 2, bufA)
        issue_gather(bufA)
        wait_gather(bufB)
        compute_scatter(bufB)

        @pl.when(h < npairs - 1)
        def _():
            issue_loads(c + 3, bufB)

        return carry

    lax.fori_loop(0, npairs, pair, 0)
    wait_gather(bufA)
    compute_scatter(bufA)
    plsc.subcore_barrier()

    pltpu.sync_copy(agg_sh.at[pl.ds(r0, _ROWS_PER_SUB)],
                    out_hbm.at[pl.ds(cid * _N + r0, _ROWS_PER_SUB)])

    @pl.when(sid == 15)
    def _():
        pltpu.sync_copy(agg_sh.at[pl.ds(tail, _N - tail)],
                        out_hbm.at[pl.ds(cid * _N + tail, _N - tail)])


def _perm_idx():
    idx = []
    for g in range(4):
        for i in range(16):
            idx.append(g * 32 + i)
            idx.append(g * 32 + 16 + i)
    return tuple(idx)


_PERM = _perm_idx()


def _gine_sc(x, ea, src, dst):
    c = _C
    mesh = plsc.VectorSubcoreMesh(core_axis_name="c", subcore_axis_name="s")
    fn = functools.partial(
        pl.kernel,
        mesh=mesh,
        out_type=jax.ShapeDtypeStruct((2 * _N, c), jnp.float32),
        scratch_types=[
            pltpu.VMEM((_CH,), jnp.int32),
            pltpu.VMEM((_CH,), jnp.int32),
            pltpu.VMEM((_CH, c), jnp.float32),
            pltpu.VMEM((_CH, c), jnp.float32),
            pltpu.VMEM((_CH, c), jnp.float32),
            pltpu.VMEM((_CH,), jnp.int32),
            pltpu.VMEM((_CH,), jnp.int32),
            pltpu.VMEM((_CH, c), jnp.float32),
            pltpu.VMEM((_CH, c), jnp.float32),
            pltpu.VMEM((_CH, c), jnp.float32),
            pltpu.VMEM_SHARED((_N, c), jnp.float32),
            pltpu.SemaphoreType.DMA,
            pltpu.SemaphoreType.DMA,
            pltpu.SemaphoreType.DMA,
            pltpu.SemaphoreType.DMA,
        ],
    )(_gine_body)
    zeros = jnp.zeros((_N, c), jnp.float32)
    return fn(x, ea, src, dst, zeros)


def _gps_layer(x, ea, src, dst, p, heads):
    aggs = _gine_sc(x, ea, src, dst)
    q, k, v = _mm3(x, (p["Wq"], p["Wk"], p["Wv"]), (p["bq"], p["bk"], p["bv"]))
    ao = _attention(q, k, v, heads)
    t2, s2 = _mm(ao, p["Wo"], p["bo"], post_adds=(x,), out_stats=True)
    han = _bn_apply(t2, s2, p["n2_g"], p["n2_b"])

    y1 = _mm(x, p["W1"], p["b1"], act="relu",
             extra_adds=(aggs[:_N], aggs[_N:]))
    t1, s1 = _mm(y1, p["W2"], p["b2"], post_adds=(x,), out_stats=True)
    hn = _bn_apply(t1, s1, p["n1_g"], p["n1_b"])

    y, out = _mm(hn, p["mW1"], p["mb1"], act="relu", extra_adds=(han,),
                 out_sum=True)
    t3, s3 = _mm(y, p["mW2"], p["mb2"], post_adds=(out,), out_stats=True)
    return _bn_apply(t3, s3, p["n3_g"], p["n3_b"], out_stats=True)


def kernel(x, edge_attr, params, edge_index):
    p1, p2 = params["gps1"], params["gps2"]
    src = edge_index[0]
    dst = edge_index[1]
    ea1 = _mm(edge_attr, p1["We"], p1["be"], block_rows=4000)
    ea2 = _mm(edge_attr, p2["We"], p2["be"], block_rows=4000)
    g1, sg1 = _gps_layer(x, ea1, src, dst, p1, heads=2)
    l1, sl1 = _mm(g1, params["lin1_W"], params["lin1_b"],
                  pre_bn=(sg1, params["bn1_g"], params["bn1_b"], True),
                  out_stats=True)
    h = _bn_apply(l1, sl1, params["bn2_g"], params["bn2_b"], relu=True)
    g2, sg2 = _gps_layer(h, ea2, src, dst, p2, heads=1)
    return _mm(g2, params["lin2_W"], params["lin2_b"],
               pre_bn=(sg2, params["bn2_g"], params["bn2_b"], True))


# speedup vs baseline: 2.4207x; 1.0226x over previous
import functools

import jax
import jax.numpy as jnp
from jax import lax
from jax.experimental import pallas as pl
from jax.experimental.pallas import tpu as pltpu
from jax.experimental.pallas import tpu_sc as plsc

_N = 10000
_E = 320000
_C = 128


def _bn_math(x, s, g, b, n_rows, relu):
    mu = s[0:1, :] / n_rows
    var = s[1:2, :] / n_rows - mu * mu
    rstd = lax.rsqrt(var + 1e-5)
    y = (x - mu) * (rstd * g) + b
    if relu:
        y = jnp.maximum(y, 0.0)
    return y


def _acc_stats(s_ref, y):
    @pl.when(pl.program_id(0) == 0)
    def _():
        s_ref[...] = jnp.zeros_like(s_ref)

    ps = jnp.sum(y, axis=0, keepdims=True)
    pq = jnp.sum(y * y, axis=0, keepdims=True)
    pad = jnp.zeros((6, y.shape[1]), jnp.float32)
    s_ref[...] += jnp.concatenate([ps, pq, pad], axis=0)


def _mm_body(nadd, npost, act, two_out, has_bn, bn_relu, has_bn2, out_stats,
             n_rows, *refs):
    i = 0
    x = refs[i][...]
    i += 1
    addvals = []
    for _ in range(nadd):
        addvals.append(refs[i][...])
        i += 1
    if has_bn:
        x = _bn_math(x, refs[i][...], refs[i + 1][...], refs[i + 2][...],
                     n_rows, bn_relu)
        i += 3
    if has_bn2:
        addvals[0] = _bn_math(addvals[0], refs[i][...], refs[i + 1][...],
                              refs[i + 2][...], n_rows, False)
        i += 3
    for a in addvals:
        x = x + a
    w = refs[i][...]
    b = refs[i + 1][...]
    i += 2
    y = jnp.dot(x, w, preferred_element_type=jnp.float32) + b
    if act == "relu":
        y = jnp.maximum(y, 0.0)
    for _ in range(npost):
        y = y + refs[i][...]
        i += 1
    refs[i][...] = y.astype(refs[i].dtype)
    i += 1
    if two_out:
        refs[i][...] = x
        i += 1
    if out_stats:
        _acc_stats(refs[i], y)


def _mm(x, w, b, act=None, extra_adds=(), post_adds=(), pre_bn=None,
        pre_bn2=None, block_rows=1000, out_sum=False, out_stats=False,
        out_dtype=jnp.float32):
    n, k = x.shape
    m = w.shape[1]
    nadd = len(extra_adds)
    npost = len(post_adds)
    grid = (n // block_rows,)
    row_spec = pl.BlockSpec((block_rows, k), lambda i: (i, 0))
    out_spec = pl.BlockSpec((block_rows, m), lambda i: (i, 0))
    stat_spec = pl.BlockSpec((8, m), lambda i: (0, 0))
    vec_spec = pl.BlockSpec((1, m), lambda i: (0, 0))
    in_specs = [row_spec] * (1 + nadd)
    args = [x, *extra_adds]
    if pre_bn is not None:
        s, g, bb, bn_relu = pre_bn
        in_specs += [pl.BlockSpec((8, k), lambda i: (0, 0)),
                     pl.BlockSpec((1, k), lambda i: (0, 0)),
                     pl.BlockSpec((1, k), lambda i: (0, 0))]
        args += [s, g.reshape(1, k), bb.reshape(1, k)]
    else:
        bn_relu = False
    if pre_bn2 is not None:
        s2, g2, bb2 = pre_bn2
        in_specs += [pl.BlockSpec((8, k), lambda i: (0, 0)),
                     pl.BlockSpec((1, k), lambda i: (0, 0)),
                     pl.BlockSpec((1, k), lambda i: (0, 0))]
        args += [s2, g2.reshape(1, k), bb2.reshape(1, k)]
    in_specs += [pl.BlockSpec((k, m), lambda i: (0, 0)), vec_spec]
    args += [w, b.reshape(1, m)]
    in_specs += [out_spec] * npost
    args += list(post_adds)
    out_shape = [jax.ShapeDtypeStruct((n, m), out_dtype)]
    out_specs = [out_spec]
    if out_sum:
        out_shape.append(jax.ShapeDtypeStruct((n, k), jnp.float32))
        out_specs.append(row_spec)
    if out_stats:
        out_shape.append(jax.ShapeDtypeStruct((8, m), jnp.float32))
        out_specs.append(stat_spec)
    fn = pl.pallas_call(
        functools.partial(_mm_body, nadd, npost, act, out_sum,
                          pre_bn is not None, bn_relu, pre_bn2 is not None,
                          out_stats, float(n)),
        grid=grid,
        in_specs=in_specs,
        out_specs=tuple(out_specs),
        out_shape=tuple(out_shape),
    )
    out = fn(*args)
    return out if (out_sum or out_stats) else out[0]


def _mm3_body(x_ref, w_ref, b_ref, o1_ref, o2_ref, o3_ref):
    y = jnp.dot(x_ref[...], w_ref[...],
                preferred_element_type=jnp.float32) + b_ref[...]
    m = o1_ref.shape[1]
    o1_ref[...] = y[:, :m]
    o2_ref[...] = y[:, m:2 * m]
    o3_ref[...] = y[:, 2 * m:]


def _mm3(x, ws, bs, block_rows=1000):
    n, k = x.shape
    m = ws[0].shape[1]
    w3 = jnp.concatenate(ws, axis=1)
    b3 = jnp.concatenate(bs).reshape(1, 3 * m)
    grid = (n // block_rows,)
    out_spec = pl.BlockSpec((block_rows, m), lambda i: (i, 0))
    fn = pl.pallas_call(
        _mm3_body,
        grid=grid,
        in_specs=[pl.BlockSpec((block_rows, k), lambda i: (i, 0)),
                  pl.BlockSpec((k, 3 * m), lambda i: (0, 0)),
                  pl.BlockSpec((1, 3 * m), lambda i: (0, 0))],
        out_specs=(out_spec, out_spec, out_spec),
        out_shape=tuple(jax.ShapeDtypeStruct((n, m), jnp.float32)
                        for _ in range(3)),
    )
    return fn(x, w3, b3)


def _stats_body(has_resid, *refs):
    if has_resid:
        t = refs[0][...] + refs[1][...]
        refs[2][...] = t
        s_ref = refs[3]
    else:
        t = refs[0][...]
        s_ref = refs[1]

    @pl.when(pl.program_id(0) == 0)
    def _():
        s_ref[...] = jnp.zeros_like(s_ref)

    ps = jnp.sum(t, axis=0, keepdims=True)
    pq = jnp.sum(t * t, axis=0, keepdims=True)
    pad = jnp.zeros((6, t.shape[1]), jnp.float32)
    s_ref[...] += jnp.concatenate([ps, pq, pad], axis=0)


def _add_stats(a, r=None, block_rows=1000):
    n, c = a.shape
    grid = (n // block_rows,)
    row_spec = pl.BlockSpec((block_rows, c), lambda i: (i, 0))
    s_spec = pl.BlockSpec((8, c), lambda i: (0, 0))
    s_shape = jax.ShapeDtypeStruct((8, c), jnp.float32)
    if r is None:
        fn = pl.pallas_call(
            functools.partial(_stats_body, False),
            grid=grid,
            in_specs=[row_spec],
            out_specs=s_spec,
            out_shape=s_shape,
        )
        return a, fn(a)
    fn = pl.pallas_call(
        functools.partial(_stats_body, True),
        grid=grid,
        in_specs=[row_spec, row_spec],
        out_specs=(row_spec, s_spec),
        out_shape=(jax.ShapeDtypeStruct((n, c), jnp.float32), s_shape),
    )
    return fn(a, r)


def _bn_body(relu, n, out_stats, *refs):
    x_ref, s_ref, g_ref, b_ref, o_ref = refs[:5]
    y = _bn_math(x_ref[...], s_ref[...], g_ref[...], b_ref[...], n, relu)
    o_ref[...] = y
    if out_stats:
        _acc_stats(refs[5], y)


def _bn_apply(x, s, g, b, relu=False, out_stats=False, block_rows=1000):
    n, c = x.shape
    grid = (n // block_rows,)
    row_spec = pl.BlockSpec((block_rows, c), lambda i: (i, 0))
    vec_spec = pl.BlockSpec((1, c), lambda i: (0, 0))
    stat_spec = pl.BlockSpec((8, c), lambda i: (0, 0))
    out_specs = [row_spec]
    out_shape = [jax.ShapeDtypeStruct((n, c), jnp.float32)]
    if out_stats:
        out_specs.append(stat_spec)
        out_shape.append(jax.ShapeDtypeStruct((8, c), jnp.float32))
    fn = pl.pallas_call(
        lambda *refs: _bn_body(relu, float(n), out_stats, *refs),
        grid=grid,
        in_specs=[row_spec, stat_spec, vec_spec, vec_spec],
        out_specs=tuple(out_specs),
        out_shape=tuple(out_shape),
    )
    out = fn(x, s, g.reshape(1, c), b.reshape(1, c))
    return out if out_stats else out[0]


def _attn_body(scale, q_ref, k_ref, v_ref, o_ref):
    q = q_ref[0].astype(jnp.bfloat16)
    k = k_ref[0].astype(jnp.bfloat16)
    s = lax.dot_general(q, k, (((1,), (1,)), ((), ())),
                        preferred_element_type=jnp.float32) * scale
    m = jnp.max(s, axis=1, keepdims=True)
    p = jnp.exp(s - m)
    denom = jnp.sum(p, axis=1, keepdims=True)
    o = lax.dot_general(p.astype(jnp.bfloat16),
                        v_ref[0].astype(jnp.bfloat16),
                        (((1,), (0,)), ((), ())),
                        preferred_element_type=jnp.float32)
    o_ref[0] = o / denom


def _attention(q, k, v, heads, block_q=400):
    n, c = q.shape
    dh = c // heads
    scale = float(dh) ** -0.5
    qh = q.reshape(n, heads, dh).transpose(1, 0, 2)
    kh = k.reshape(n, heads, dh).transpose(1, 0, 2)
    vh = v.reshape(n, heads, dh).transpose(1, 0, 2)
    grid = (heads, n // block_q)
    q_spec = pl.BlockSpec((1, block_q, dh), lambda h, i: (h, i, 0))
    kv_spec = pl.BlockSpec((1, n, dh), lambda h, i: (h, 0, 0))
    fn = pl.pallas_call(
        functools.partial(_attn_body, scale),
        grid=grid,
        in_specs=[q_spec, kv_spec, kv_spec],
        out_specs=q_spec,
        out_shape=jax.ShapeDtypeStruct((heads, n, dh), jnp.float32),
    )
    oh = fn(qh, kh, vh)
    return oh.transpose(1, 0, 2).reshape(n, c)


_CH = 80
_ROWS_PER_SUB = 624


def _gine_body(x_hbm, ea_hbm, src_hbm, dst_hbm, z_hbm, out_hbm,
               sidx0, didx0, rows0, eab0, sidx1, didx1, rows1, eab1,
               agg_sh, lsem0, gsem0, ssem0, lsem1, gsem1, ssem1):
    cid = lax.axis_index("c")
    sid = lax.axis_index("s")
    wid = sid * 2 + cid
    e0 = wid * (_E // 32)
    bufA = (sidx0, didx0, rows0, eab0, lsem0, gsem0, ssem0)
    bufB = (sidx1, didx1, rows1, eab1, lsem1, gsem1, ssem1)

    def issue_loads(c, bs):
        sidx, didx, rows, eab, lsem, gsem, ssem = bs
        base = e0 + c * _CH
        pltpu.async_copy(src_hbm.at[pl.ds(base, _CH)], sidx, lsem)
        pltpu.async_copy(dst_hbm.at[pl.ds(base, _CH)], didx, lsem)
        pltpu.async_copy(ea_hbm.at[pl.ds(base, _CH)], eab, lsem)

    def wait_loads(c, bs):
        sidx, didx, rows, eab, lsem, gsem, ssem = bs
        base = e0 + c * _CH
        pltpu.make_async_copy(src_hbm.at[pl.ds(base, _CH)], sidx, lsem).wait()
        pltpu.make_async_copy(dst_hbm.at[pl.ds(base, _CH)], didx, lsem).wait()
        pltpu.make_async_copy(ea_hbm.at[pl.ds(base, _CH)], eab, lsem).wait()

    def issue_gather(bs):
        sidx, didx, rows, eab, lsem, gsem, ssem = bs
        pltpu.async_copy(x_hbm.at[sidx], rows, gsem)

    def wait_gather(bs):
        sidx, didx, rows, eab, lsem, gsem, ssem = bs
        pltpu.make_async_copy(x_hbm.at[sidx], rows, gsem).wait()

    def compute_scatter(bs):
        sidx, didx, rows, eab, lsem, gsem, ssem = bs

        def edge4(i, carry):
            for u in range(4):
                r = i * 4 + u
                for j in range(8):
                    sl = pl.ds(j * 16, 16)
                    rows[r, sl] = jnp.maximum(rows[r, sl] + eab[r, sl], 0.0)
            return carry

        lax.fori_loop(0, _CH // 4, edge4, 0)
        pltpu.async_copy(rows, agg_sh.at[didx], ssem, add=True)

    def wait_scatter(bs):
        sidx, didx, rows, eab, lsem, gsem, ssem = bs
        pltpu.make_async_copy(rows, agg_sh.at[didx], ssem).wait()

    r0 = sid * _ROWS_PER_SUB
    tail = 16 * _ROWS_PER_SUB
    pltpu.sync_copy(z_hbm.at[pl.ds(r0, _ROWS_PER_SUB)],
                    agg_sh.at[pl.ds(r0, _ROWS_PER_SUB)])

    @pl.when(sid == 15)
    def _():
        pltpu.sync_copy(z_hbm.at[pl.ds(tail, _N - tail)],
                        agg_sh.at[pl.ds(tail, _N - tail)])

    plsc.subcore_barrier()

    nchunks = (_E // 32) // _CH
    npairs = (nchunks - 1) // 2

    issue_loads(0, bufA)
    wait_loads(0, bufA)
    issue_gather(bufA)
    issue_loads(1, bufB)

    def pair(h, carry):
        c = 2 * h
        wait_loads(c + 1, bufB)
        issue_gather(bufB)
        wait_gather(bufA)
        compute_scatter(bufA)
        wait_gather(bufB)
        compute_scatter(bufB)
        wait_scatter(bufA)
        issue_loads(c + 2, bufA)
        wait_loads(c + 2, bufA)
        issue_gather(bufA)

        @pl.when(h < npairs - 1)
        def _():
            wait_scatter(bufB)
            issue_loads(c + 3, bufB)

        return carry

    lax.fori_loop(0, npairs, pair, 0)
    wait_gather(bufA)
    wait_scatter(bufB)
    compute_scatter(bufA)
    wait_scatter(bufA)
    plsc.subcore_barrier()

    pltpu.sync_copy(agg_sh.at[pl.ds(r0, _ROWS_PER_SUB)],
                    out_hbm.at[pl.ds(cid * _N + r0, _ROWS_PER_SUB)])

    @pl.when(sid == 15)
    def _():
        pltpu.sync_copy(agg_sh.at[pl.ds(tail, _N - tail)],
                        out_hbm.at[pl.ds(cid * _N + tail, _N - tail)])


def _perm_idx():
    idx = []
    for g in range(4):
        for i in range(16):
            idx.append(g * 32 + i)
            idx.append(g * 32 + 16 + i)
    return tuple(idx)


_PERM = _perm_idx()


def _gine_sc(x, ea, src, dst):
    c = _C
    mesh = plsc.VectorSubcoreMesh(core_axis_name="c", subcore_axis_name="s")
    fn = functools.partial(
        pl.kernel,
        mesh=mesh,
        out_type=jax.ShapeDtypeStruct((2 * _N, c), jnp.float32),
        scratch_types=[
            pltpu.VMEM((_CH,), jnp.int32),
            pltpu.VMEM((_CH,), jnp.int32),
            pltpu.VMEM((_CH, c), jnp.float32),
            pltpu.VMEM((_CH, c), jnp.float32),
            pltpu.VMEM((_CH,), jnp.int32),
            pltpu.VMEM((_CH,), jnp.int32),
            pltpu.VMEM((_CH, c), jnp.float32),
            pltpu.VMEM((_CH, c), jnp.float32),
            pltpu.VMEM_SHARED((_N, c), jnp.float32),
            pltpu.SemaphoreType.DMA,
            pltpu.SemaphoreType.DMA,
            pltpu.SemaphoreType.DMA,
            pltpu.SemaphoreType.DMA,
            pltpu.SemaphoreType.DMA,
            pltpu.SemaphoreType.DMA,
        ],
    )(_gine_body)
    zeros = jnp.zeros((_N, c), jnp.float32)
    return fn(x, ea, src, dst, zeros)


def _gps_layer(x, ea, src, dst, p, heads):
    aggs = _gine_sc(x, ea, src, dst)
    q, k, v = _mm3(x, (p["Wq"], p["Wk"], p["Wv"]), (p["bq"], p["bk"], p["bv"]))
    ao = _attention(q, k, v, heads)
    t2, s2 = _mm(ao, p["Wo"], p["bo"], post_adds=(x,), out_stats=True)

    y1 = _mm(x, p["W1"], p["b1"], act="relu",
             extra_adds=(aggs[:_N], aggs[_N:]))
    t1, s1 = _mm(y1, p["W2"], p["b2"], post_adds=(x,), out_stats=True)

    y, out = _mm(t1, p["mW1"], p["mb1"], act="relu", extra_adds=(t2,),
                 pre_bn=(s1, p["n1_g"], p["n1_b"], False),
                 pre_bn2=(s2, p["n2_g"], p["n2_b"]),
                 out_sum=True)
    t3, s3 = _mm(y, p["mW2"], p["mb2"], post_adds=(out,), out_stats=True)
    return _bn_apply(t3, s3, p["n3_g"], p["n3_b"], out_stats=True)


def kernel(x, edge_attr, params, edge_index):
    p1, p2 = params["gps1"], params["gps2"]
    src = edge_index[0]
    dst = edge_index[1]
    ea1 = _mm(edge_attr, p1["We"], p1["be"], block_rows=4000)
    ea2 = _mm(edge_attr, p2["We"], p2["be"], block_rows=4000)
    g1, sg1 = _gps_layer(x, ea1, src, dst, p1, heads=2)
    l1, sl1 = _mm(g1, params["lin1_W"], params["lin1_b"],
                  pre_bn=(sg1, params["bn1_g"], params["bn1_b"], True),
                  out_stats=True)
    h = _bn_apply(l1, sl1, params["bn2_g"], params["bn2_b"], relu=True)
    g2, sg2 = _gps_layer(h, ea2, src, dst, p2, heads=1)
    return _mm(g2, params["lin2_W"], params["lin2_b"],
               pre_bn=(sg2, params["bn2_g"], params["bn2_b"], True))


# speedup vs baseline: 2.5461x; 1.0518x over previous
import functools

import jax
import jax.numpy as jnp
from jax import lax
from jax.experimental import pallas as pl
from jax.experimental.pallas import tpu as pltpu
from jax.experimental.pallas import tpu_sc as plsc

_N = 10000
_E = 320000
_C = 128


def _bn_math(x, s, g, b, n_rows, relu):
    mu = s[0:1, :] / n_rows
    var = s[1:2, :] / n_rows - mu * mu
    rstd = lax.rsqrt(var + 1e-5)
    y = (x - mu) * (rstd * g) + b
    if relu:
        y = jnp.maximum(y, 0.0)
    return y


def _acc_stats(s_ref, y):
    @pl.when(pl.program_id(0) == 0)
    def _():
        s_ref[...] = jnp.zeros_like(s_ref)

    ps = jnp.sum(y, axis=0, keepdims=True)
    pq = jnp.sum(y * y, axis=0, keepdims=True)
    pad = jnp.zeros((6, y.shape[1]), jnp.float32)
    s_ref[...] += jnp.concatenate([ps, pq, pad], axis=0)


def _mm_body(nadd, npost, act, two_out, has_bn, bn_relu, has_bn2, out_stats,
             n_rows, *refs):
    i = 0
    x = refs[i][...]
    i += 1
    addvals = []
    for _ in range(nadd):
        addvals.append(refs[i][...])
        i += 1
    if has_bn:
        x = _bn_math(x, refs[i][...], refs[i + 1][...], refs[i + 2][...],
                     n_rows, bn_relu)
        i += 3
    if has_bn2:
        addvals[0] = _bn_math(addvals[0], refs[i][...], refs[i + 1][...],
                              refs[i + 2][...], n_rows, False)
        i += 3
    for a in addvals:
        x = x + a
    w = refs[i][...]
    b = refs[i + 1][...]
    i += 2
    y = jnp.dot(x, w, preferred_element_type=jnp.float32) + b
    if act == "relu":
        y = jnp.maximum(y, 0.0)
    for _ in range(npost):
        y = y + refs[i][...]
        i += 1
    refs[i][...] = y.astype(refs[i].dtype)
    i += 1
    if two_out:
        refs[i][...] = x
        i += 1
    if out_stats:
        _acc_stats(refs[i], y)


def _mm(x, w, b, act=None, extra_adds=(), post_adds=(), pre_bn=None,
        pre_bn2=None, block_rows=1000, out_sum=False, out_stats=False,
        out_dtype=jnp.float32):
    n, k = x.shape
    m = w.shape[1]
    nadd = len(extra_adds)
    npost = len(post_adds)
    grid = (n // block_rows,)
    row_spec = pl.BlockSpec((block_rows, k), lambda i: (i, 0))
    out_spec = pl.BlockSpec((block_rows, m), lambda i: (i, 0))
    stat_spec = pl.BlockSpec((8, m), lambda i: (0, 0))
    vec_spec = pl.BlockSpec((1, m), lambda i: (0, 0))
    in_specs = [row_spec] * (1 + nadd)
    args = [x, *extra_adds]
    if pre_bn is not None:
        s, g, bb, bn_relu = pre_bn
        in_specs += [pl.BlockSpec((8, k), lambda i: (0, 0)),
                     pl.BlockSpec((1, k), lambda i: (0, 0)),
                     pl.BlockSpec((1, k), lambda i: (0, 0))]
        args += [s, g.reshape(1, k), bb.reshape(1, k)]
    else:
        bn_relu = False
    if pre_bn2 is not None:
        s2, g2, bb2 = pre_bn2
        in_specs += [pl.BlockSpec((8, k), lambda i: (0, 0)),
                     pl.BlockSpec((1, k), lambda i: (0, 0)),
                     pl.BlockSpec((1, k), lambda i: (0, 0))]
        args += [s2, g2.reshape(1, k), bb2.reshape(1, k)]
    in_specs += [pl.BlockSpec((k, m), lambda i: (0, 0)), vec_spec]
    args += [w, b.reshape(1, m)]
    in_specs += [out_spec] * npost
    args += list(post_adds)
    out_shape = [jax.ShapeDtypeStruct((n, m), out_dtype)]
    out_specs = [out_spec]
    if out_sum:
        out_shape.append(jax.ShapeDtypeStruct((n, k), jnp.float32))
        out_specs.append(row_spec)
    if out_stats:
        out_shape.append(jax.ShapeDtypeStruct((8, m), jnp.float32))
        out_specs.append(stat_spec)
    fn = pl.pallas_call(
        functools.partial(_mm_body, nadd, npost, act, out_sum,
                          pre_bn is not None, bn_relu, pre_bn2 is not None,
                          out_stats, float(n)),
        grid=grid,
        in_specs=in_specs,
        out_specs=tuple(out_specs),
        out_shape=tuple(out_shape),
    )
    out = fn(*args)
    return out if (out_sum or out_stats) else out[0]


def _mmn_body(nout, x_ref, w_ref, b_ref, *o_refs):
    y = jnp.dot(x_ref[...], w_ref[...],
                preferred_element_type=jnp.float32) + b_ref[...]
    m = o_refs[0].shape[1]
    for t in range(nout):
        o_refs[t][...] = y[:, t * m:(t + 1) * m]


def _mm3(x, ws, bs, block_rows=1000):
    n, k = x.shape
    m = ws[0].shape[1]
    nout = len(ws)
    w3 = jnp.concatenate(ws, axis=1)
    b3 = jnp.concatenate(bs).reshape(1, nout * m)
    grid = (n // block_rows,)
    out_spec = pl.BlockSpec((block_rows, m), lambda i: (i, 0))
    fn = pl.pallas_call(
        functools.partial(_mmn_body, nout),
        grid=grid,
        in_specs=[pl.BlockSpec((block_rows, k), lambda i: (i, 0)),
                  pl.BlockSpec((k, nout * m), lambda i: (0, 0)),
                  pl.BlockSpec((1, nout * m), lambda i: (0, 0))],
        out_specs=(out_spec,) * nout,
        out_shape=tuple(jax.ShapeDtypeStruct((n, m), jnp.float32)
                        for _ in range(nout)),
    )
    return fn(x, w3, b3)


def _stats_body(has_resid, *refs):
    if has_resid:
        t = refs[0][...] + refs[1][...]
        refs[2][...] = t
        s_ref = refs[3]
    else:
        t = refs[0][...]
        s_ref = refs[1]

    @pl.when(pl.program_id(0) == 0)
    def _():
        s_ref[...] = jnp.zeros_like(s_ref)

    ps = jnp.sum(t, axis=0, keepdims=True)
    pq = jnp.sum(t * t, axis=0, keepdims=True)
    pad = jnp.zeros((6, t.shape[1]), jnp.float32)
    s_ref[...] += jnp.concatenate([ps, pq, pad], axis=0)


def _add_stats(a, r=None, block_rows=1000):
    n, c = a.shape
    grid = (n // block_rows,)
    row_spec = pl.BlockSpec((block_rows, c), lambda i: (i, 0))
    s_spec = pl.BlockSpec((8, c), lambda i: (0, 0))
    s_shape = jax.ShapeDtypeStruct((8, c), jnp.float32)
    if r is None:
        fn = pl.pallas_call(
            functools.partial(_stats_body, False),
            grid=grid,
            in_specs=[row_spec],
            out_specs=s_spec,
            out_shape=s_shape,
        )
        return a, fn(a)
    fn = pl.pallas_call(
        functools.partial(_stats_body, True),
        grid=grid,
        in_specs=[row_spec, row_spec],
        out_specs=(row_spec, s_spec),
        out_shape=(jax.ShapeDtypeStruct((n, c), jnp.float32), s_shape),
    )
    return fn(a, r)


def _bn_body(relu, n, out_stats, *refs):
    x_ref, s_ref, g_ref, b_ref, o_ref = refs[:5]
    y = _bn_math(x_ref[...], s_ref[...], g_ref[...], b_ref[...], n, relu)
    o_ref[...] = y
    if out_stats:
        _acc_stats(refs[5], y)


def _bn_apply(x, s, g, b, relu=False, out_stats=False, block_rows=1000):
    n, c = x.shape
    grid = (n // block_rows,)
    row_spec = pl.BlockSpec((block_rows, c), lambda i: (i, 0))
    vec_spec = pl.BlockSpec((1, c), lambda i: (0, 0))
    stat_spec = pl.BlockSpec((8, c), lambda i: (0, 0))
    out_specs = [row_spec]
    out_shape = [jax.ShapeDtypeStruct((n, c), jnp.float32)]
    if out_stats:
        out_specs.append(stat_spec)
        out_shape.append(jax.ShapeDtypeStruct((8, c), jnp.float32))
    fn = pl.pallas_call(
        lambda *refs: _bn_body(relu, float(n), out_stats, *refs),
        grid=grid,
        in_specs=[row_spec, stat_spec, vec_spec, vec_spec],
        out_specs=tuple(out_specs),
        out_shape=tuple(out_shape),
    )
    out = fn(x, s, g.reshape(1, c), b.reshape(1, c))
    return out if out_stats else out[0]


def _attn_body(scale, q_ref, k_ref, v_ref, o_ref):
    q = q_ref[0].astype(jnp.bfloat16)
    k = k_ref[0].astype(jnp.bfloat16)
    s = lax.dot_general(q, k, (((1,), (1,)), ((), ())),
                        preferred_element_type=jnp.float32) * scale
    m = jnp.max(s, axis=1, keepdims=True)
    p = jnp.exp(s - m)
    denom = jnp.sum(p, axis=1, keepdims=True)
    o = lax.dot_general(p.astype(jnp.bfloat16),
                        v_ref[0].astype(jnp.bfloat16),
                        (((1,), (0,)), ((), ())),
                        preferred_element_type=jnp.float32)
    o_ref[0] = o / denom


def _attention(q, k, v, heads, block_q=400):
    n, c = q.shape
    dh = c // heads
    scale = float(dh) ** -0.5
    qh = q.reshape(n, heads, dh).transpose(1, 0, 2)
    kh = k.reshape(n, heads, dh).transpose(1, 0, 2)
    vh = v.reshape(n, heads, dh).transpose(1, 0, 2)
    grid = (heads, n // block_q)
    q_spec = pl.BlockSpec((1, block_q, dh), lambda h, i: (h, i, 0))
    kv_spec = pl.BlockSpec((1, n, dh), lambda h, i: (h, 0, 0))
    fn = pl.pallas_call(
        functools.partial(_attn_body, scale),
        grid=grid,
        in_specs=[q_spec, kv_spec, kv_spec],
        out_specs=q_spec,
        out_shape=jax.ShapeDtypeStruct((heads, n, dh), jnp.float32),
    )
    oh = fn(qh, kh, vh)
    return oh.transpose(1, 0, 2).reshape(n, c)


_CH = 80
_ROWS_PER_SUB = 624


def _gine_body(x_hbm, ea_hbm, src_hbm, dst_hbm, z_hbm, out_hbm,
               sidx0, didx0, rows0, eab0, sidx1, didx1, rows1, eab1,
               agg_sh, lsem0, gsem0, ssem0, lsem1, gsem1, ssem1):
    cid = lax.axis_index("c")
    sid = lax.axis_index("s")
    wid = sid * 2 + cid
    e0 = wid * (_E // 32)
    bufA = (sidx0, didx0, rows0, eab0, lsem0, gsem0, ssem0)
    bufB = (sidx1, didx1, rows1, eab1, lsem1, gsem1, ssem1)

    def issue_loads(c, bs):
        sidx, didx, rows, eab, lsem, gsem, ssem = bs
        base = e0 + c * _CH
        pltpu.async_copy(src_hbm.at[pl.ds(base, _CH)], sidx, lsem)
        pltpu.async_copy(dst_hbm.at[pl.ds(base, _CH)], didx, lsem)
        pltpu.async_copy(ea_hbm.at[pl.ds(base, _CH)], eab, lsem)

    def wait_loads(c, bs):
        sidx, didx, rows, eab, lsem, gsem, ssem = bs
        base = e0 + c * _CH
        pltpu.make_async_copy(src_hbm.at[pl.ds(base, _CH)], sidx, lsem).wait()
        pltpu.make_async_copy(dst_hbm.at[pl.ds(base, _CH)], didx, lsem).wait()
        pltpu.make_async_copy(ea_hbm.at[pl.ds(base, _CH)], eab, lsem).wait()

    def issue_gather(bs):
        sidx, didx, rows, eab, lsem, gsem, ssem = bs
        pltpu.async_copy(x_hbm.at[sidx], rows, gsem)

    def wait_gather(bs):
        sidx, didx, rows, eab, lsem, gsem, ssem = bs
        pltpu.make_async_copy(x_hbm.at[sidx], rows, gsem).wait()

    def compute_scatter(bs):
        sidx, didx, rows, eab, lsem, gsem, ssem = bs

        def edge4(i, carry):
            for u in range(4):
                r = i * 4 + u
                for j in range(8):
                    sl = pl.ds(j * 16, 16)
                    rows[r, sl] = jnp.maximum(rows[r, sl] + eab[r, sl], 0.0)
            return carry

        lax.fori_loop(0, _CH // 4, edge4, 0)
        pltpu.async_copy(rows, agg_sh.at[didx], ssem, add=True)

    def wait_scatter(bs):
        sidx, didx, rows, eab, lsem, gsem, ssem = bs
        pltpu.make_async_copy(rows, agg_sh.at[didx], ssem).wait()

    nchunks = (_E // 32) // _CH
    npairs = (nchunks - 1) // 2

    issue_loads(0, bufA)
    issue_loads(1, bufB)

    r0 = sid * _ROWS_PER_SUB
    tail = 16 * _ROWS_PER_SUB
    pltpu.sync_copy(z_hbm.at[pl.ds(r0, _ROWS_PER_SUB)],
                    agg_sh.at[pl.ds(r0, _ROWS_PER_SUB)])

    @pl.when(sid == 15)
    def _():
        pltpu.sync_copy(z_hbm.at[pl.ds(tail, _N - tail)],
                        agg_sh.at[pl.ds(tail, _N - tail)])

    plsc.subcore_barrier()

    wait_loads(0, bufA)
    issue_gather(bufA)

    def pair(h, carry):
        c = 2 * h
        wait_loads(c + 1, bufB)
        issue_gather(bufB)
        wait_gather(bufA)
        compute_scatter(bufA)
        wait_gather(bufB)
        compute_scatter(bufB)
        wait_scatter(bufA)
        issue_loads(c + 2, bufA)
        wait_loads(c + 2, bufA)
        issue_gather(bufA)

        @pl.when(h < npairs - 1)
        def _():
            wait_scatter(bufB)
            issue_loads(c + 3, bufB)

        return carry

    lax.fori_loop(0, npairs, pair, 0)
    wait_gather(bufA)
    wait_scatter(bufB)
    compute_scatter(bufA)
    wait_scatter(bufA)
    plsc.subcore_barrier()

    pltpu.sync_copy(agg_sh.at[pl.ds(r0, _ROWS_PER_SUB)],
                    out_hbm.at[pl.ds(cid * _N + r0, _ROWS_PER_SUB)])

    @pl.when(sid == 15)
    def _():
        pltpu.sync_copy(agg_sh.at[pl.ds(tail, _N - tail)],
                        out_hbm.at[pl.ds(cid * _N + tail, _N - tail)])


def _perm_idx():
    idx = []
    for g in range(4):
        for i in range(16):
            idx.append(g * 32 + i)
            idx.append(g * 32 + 16 + i)
    return tuple(idx)


_PERM = _perm_idx()


def _gine_sc(x, ea, src, dst):
    c = _C
    mesh = plsc.VectorSubcoreMesh(core_axis_name="c", subcore_axis_name="s")
    fn = functools.partial(
        pl.kernel,
        mesh=mesh,
        out_type=jax.ShapeDtypeStruct((2 * _N, c), jnp.float32),
        scratch_types=[
            pltpu.VMEM((_CH,), jnp.int32),
            pltpu.VMEM((_CH,), jnp.int32),
            pltpu.VMEM((_CH, c), jnp.float32),
            pltpu.VMEM((_CH, c), jnp.float32),
            pltpu.VMEM((_CH,), jnp.int32),
            pltpu.VMEM((_CH,), jnp.int32),
            pltpu.VMEM((_CH, c), jnp.float32),
            pltpu.VMEM((_CH, c), jnp.float32),
            pltpu.VMEM_SHARED((_N, c), jnp.float32),
            pltpu.SemaphoreType.DMA,
            pltpu.SemaphoreType.DMA,
            pltpu.SemaphoreType.DMA,
            pltpu.SemaphoreType.DMA,
            pltpu.SemaphoreType.DMA,
            pltpu.SemaphoreType.DMA,
        ],
    )(_gine_body)
    zeros = jnp.zeros((_N, c), jnp.float32)
    return fn(x, ea, src, dst, zeros)


def _gps_layer(x, ea, src, dst, p, heads):
    aggs = _gine_sc(x, ea, src, dst)
    q, k, v = _mm3(x, (p["Wq"], p["Wk"], p["Wv"]), (p["bq"], p["bk"], p["bv"]))
    ao = _attention(q, k, v, heads)
    t2, s2 = _mm(ao, p["Wo"], p["bo"], post_adds=(x,), out_stats=True)

    y1 = _mm(x, p["W1"], p["b1"], act="relu",
             extra_adds=(aggs[:_N], aggs[_N:]))
    t1, s1 = _mm(y1, p["W2"], p["b2"], post_adds=(x,), out_stats=True)

    y, out = _mm(t1, p["mW1"], p["mb1"], act="relu", extra_adds=(t2,),
                 pre_bn=(s1, p["n1_g"], p["n1_b"], False),
                 pre_bn2=(s2, p["n2_g"], p["n2_b"]),
                 out_sum=True)
    t3, s3 = _mm(y, p["mW2"], p["mb2"], post_adds=(out,), out_stats=True)
    return _bn_apply(t3, s3, p["n3_g"], p["n3_b"], out_stats=True)


def kernel(x, edge_attr, params, edge_index):
    p1, p2 = params["gps1"], params["gps2"]
    src = edge_index[0]
    dst = edge_index[1]
    ea1, ea2 = _mm3(edge_attr, (p1["We"], p2["We"]), (p1["be"], p2["be"]),
                    block_rows=4000)
    g1, sg1 = _gps_layer(x, ea1, src, dst, p1, heads=2)
    l1, sl1 = _mm(g1, params["lin1_W"], params["lin1_b"],
                  pre_bn=(sg1, params["bn1_g"], params["bn1_b"], True),
                  out_stats=True)
    h = _bn_apply(l1, sl1, params["bn2_g"], params["bn2_b"], relu=True)
    g2, sg2 = _gps_layer(h, ea2, src, dst, p2, heads=1)
    return _mm(g2, params["lin2_W"], params["lin2_b"],
               pre_bn=(sg2, params["bn2_g"], params["bn2_b"], True))
